# baseline (speedup 1.0000x reference)
import functools

import jax
import jax.numpy as jnp
from jax import lax
from jax.experimental import pallas as pl
from jax.experimental.pallas import tpu as pltpu
from jax.experimental.pallas import tpu_sc as plsc

B = 16384
F = 512
DG = 32
NV = 100000
NC = 2
NS = 16
NW = NC * NS
BPW = B // NW
CH = 16
NCHUNK = BPW // CH
SCH = 64
NSCHUNK = BPW // SCH
NBUF = 3


def _pipe_gather(base, nchunks, ch, streams, sg, sw):
    def fire(c):
        slot = c % NBUF
        isl = pl.ds(c * ch, ch)
        return [pltpu.async_copy(tab.at[idx.at[isl]], bufs[slot], sg[slot])
                for (tab, idx, bufs, _) in streams]

    def write(c):
        slot = c % NBUF
        osl = pl.ds(base + c * ch, ch)
        ws = [pltpu.make_async_copy(bufs[slot], out.at[osl], sw[slot])
              for (_, _, bufs, out) in streams]
        for w in ws:
            w.start()
        return ws

    gathers = {0: fire(0)}
    writes = {}
    for c in range(nchunks):
        if c >= NBUF - 1:
            for w in writes.pop(c - (NBUF - 1)):
                w.wait()
        if c + 1 < nchunks:
            gathers[c + 1] = fire(c + 1)
        for g in gathers.pop(c):
            g.wait()
        writes[c] = write(c)
    for c in sorted(writes):
        for w in writes.pop(c):
            w.wait()


def _sc_feat_body(pi_hbm, ni_hbm, features, pf_out, nf_out,
                  pi_v, ni_v,
                  pf0, pf1, pf2, nf0, nf1, nf2,
                  sg0, sg1, sg2, sw0, sw1, sw2):
    wid = lax.axis_index("c") * NS + lax.axis_index("s")
    base = wid * BPW
    pltpu.sync_copy(pi_hbm.at[pl.ds(base, BPW)], pi_v)
    pltpu.sync_copy(ni_hbm.at[pl.ds(base, BPW)], ni_v)
    _pipe_gather(
        base, NCHUNK, CH,
        [(features, pi_v, (pf0, pf1, pf2), pf_out),
         (features, ni_v, (nf0, nf1, nf2), nf_out)],
        (sg0, sg1, sg2), (sw0, sw1, sw2))


@functools.partial(
    pl.kernel,
    out_type=(
        jax.ShapeDtypeStruct((B, F), jnp.float32),
        jax.ShapeDtypeStruct((B, F), jnp.float32),
    ),
    mesh=plsc.VectorSubcoreMesh(core_axis_name="c", subcore_axis_name="s"),
    scratch_types=(
        [pltpu.VMEM((BPW,), jnp.int32)] * 2
        + [pltpu.VMEM((CH, F), jnp.float32)] * 6
        + [pltpu.SemaphoreType.DMA] * 6
    ),
)
def _sc_feat(*refs):
    _sc_feat_body(*refs)


def _sc_small_body(ui_hbm, pi_hbm, ni_hbm, gamma_users, gamma_items,
                   theta_users, beta_items,
                   gu_out, tu_out, gip_out, gin_out, bp_out, bn_out,
                   ui_v, pi_v, ni_v,
                   gu_v, tu_v, gip_v, gin_v, bp_v, bn_v,
                   ui_s, pi_s, ni_s, sem_s):
    wid = lax.axis_index("c") * NS + lax.axis_index("s")
    base = wid * BPW
    pltpu.sync_copy(ui_hbm.at[pl.ds(base, BPW)], ui_v)
    pltpu.sync_copy(pi_hbm.at[pl.ds(base, BPW)], pi_v)
    pltpu.sync_copy(ni_hbm.at[pl.ds(base, BPW)], ni_v)

    def small_chunk(c):
        def extract(g, _):
            st = c * SCH + g * 16
            u16 = ui_v[pl.ds(st, 16)]
            p16 = pi_v[pl.ds(st, 16)]
            n16 = ni_v[pl.ds(st, 16)]
            for l in range(16):
                row = g * 16 + l
                ui_s[row] = u16[l]
                pi_s[row] = p16[l]
                ni_s[row] = n16[l]
            return 0
        lax.fori_loop(0, SCH // 16, extract, 0)

        def fire(e, _):
            u = ui_s[e]
            p = pi_s[e]
            n = ni_s[e]
            pltpu.async_copy(gamma_users.at[pl.ds(u, 1)],
                             gu_v.at[pl.ds(e, 1)], sem_s)
            pltpu.async_copy(theta_users.at[pl.ds(u, 1)],
                             tu_v.at[pl.ds(e, 1)], sem_s)
            pltpu.async_copy(gamma_items.at[pl.ds(p, 1)],
                             gip_v.at[pl.ds(e, 1)], sem_s)
            pltpu.async_copy(gamma_items.at[pl.ds(n, 1)],
                             gin_v.at[pl.ds(e, 1)], sem_s)
            pltpu.async_copy(beta_items.at[pl.ds(p, 1)],
                             bp_v.at[pl.ds(e, 1)], sem_s)
            pltpu.async_copy(beta_items.at[pl.ds(n, 1)],
                             bn_v.at[pl.ds(e, 1)], sem_s)
            return 0
        lax.fori_loop(0, SCH, fire, 0)

        def drain(e, _):
            pltpu.make_async_copy(gamma_users.at[pl.ds(0, 1)],
                                  gu_v.at[pl.ds(e, 1)], sem_s).wait()
            pltpu.make_async_copy(theta_users.at[pl.ds(0, 1)],
                                  tu_v.at[pl.ds(e, 1)], sem_s).wait()
            pltpu.make_async_copy(gamma_items.at[pl.ds(0, 1)],
                                  gip_v.at[pl.ds(e, 1)], sem_s).wait()
            pltpu.make_async_copy(gamma_items.at[pl.ds(0, 1)],
                                  gin_v.at[pl.ds(e, 1)], sem_s).wait()
            pltpu.make_async_copy(beta_items.at[pl.ds(0, 1)],
                                  bp_v.at[pl.ds(e, 1)], sem_s).wait()
            pltpu.make_async_copy(beta_items.at[pl.ds(0, 1)],
                                  bn_v.at[pl.ds(e, 1)], sem_s).wait()
            return 0
        lax.fori_loop(0, SCH, drain, 0)

    for c in range(NSCHUNK):
        small_chunk(c)
        osl = pl.ds(base + c * SCH, SCH)
        pltpu.sync_copy(gu_v, gu_out.at[osl])
        pltpu.sync_copy(tu_v, tu_out.at[osl])
        pltpu.sync_copy(gip_v, gip_out.at[osl])
        pltpu.sync_copy(gin_v, gin_out.at[osl])
        pltpu.sync_copy(bp_v, bp_out.at[osl])
        pltpu.sync_copy(bn_v, bn_out.at[osl])


@functools.partial(
    pl.kernel,
    out_type=(
        jax.ShapeDtypeStruct((B, DG), jnp.float32),
        jax.ShapeDtypeStruct((B, DG), jnp.float32),
        jax.ShapeDtypeStruct((B, DG), jnp.float32),
        jax.ShapeDtypeStruct((B, DG), jnp.float32),
        jax.ShapeDtypeStruct((B, 1), jnp.float32),
        jax.ShapeDtypeStruct((B, 1), jnp.float32),
    ),
    mesh=plsc.VectorSubcoreMesh(core_axis_name="c", subcore_axis_name="s"),
    scratch_types=(
        [pltpu.VMEM((BPW,), jnp.int32)] * 3
        + [pltpu.VMEM((SCH, DG), jnp.float32)] * 4
        + [pltpu.VMEM((SCH, 1), jnp.float32)] * 2
        + [pltpu.SMEM((SCH,), jnp.int32)] * 3
        + [pltpu.SemaphoreType.DMA] * 1
    ),
)
def _sc_small(*refs):
    _sc_small_body(*refs)


BB = 2048


def _tc_combine_body(pf, nf, gu, tu, gip, gin, bp, bn, emb, vb, out):
    diff = pf[...] - nf[...]
    g = jnp.dot(diff, emb[...], preferred_element_type=jnp.float32)
    s_vis = jnp.sum(tu[...] * g, axis=1, keepdims=True)
    s_bias = jnp.dot(diff, vb[...], preferred_element_type=jnp.float32)
    s_lat = jnp.sum(gu[...] * (gip[...] - gin[...]), axis=1, keepdims=True)
    out[...] = bp[...] - bn[...] + s_lat + s_vis + s_bias


def _tc_combine(pf, nf, gu, tu, gip, gin, bp, bn, emb, vb):
    bspec_f = pl.BlockSpec((BB, F), lambda i: (i, 0))
    bspec_s = pl.BlockSpec((BB, DG), lambda i: (i, 0))
    bspec_1 = pl.BlockSpec((BB, 1), lambda i: (i, 0))
    return pl.pallas_call(
        _tc_combine_body,
        grid=(B // BB,),
        in_specs=[
            bspec_f, bspec_f, bspec_s, bspec_s, bspec_s, bspec_s,
            bspec_1, bspec_1,
            pl.BlockSpec((F, DG), lambda i: (0, 0)),
            pl.BlockSpec((F, 1), lambda i: (0, 0)),
        ],
        out_specs=bspec_1,
        out_shape=jax.ShapeDtypeStruct((B, 1), jnp.float32),
    )(pf, nf, gu, tu, gip, gin, bp, bn, emb, vb)[:, 0]


def kernel(ui, pi, ni, features, gamma_users, gamma_items, theta_users,
           embedding, beta_items, visual_bias):
    pf, nf = _sc_feat(pi, ni, features)
    gu, tu, gip, gin, bp, bn = _sc_small(
        ui, pi, ni, gamma_users, gamma_items, theta_users, beta_items)
    return _tc_combine(pf, nf, gu, tu, gip, gin, bp, bn, embedding, visual_bias)


# speedup vs baseline: 1.0949x; 1.0949x over previous
import functools

import jax
import jax.numpy as jnp
from jax import lax
from jax.experimental import pallas as pl
from jax.experimental.pallas import tpu as pltpu
from jax.experimental.pallas import tpu_sc as plsc

B = 16384
F = 512
DG = 32
NV = 100000
NC = 2
NS = 16
NW = NC * NS
BPW = B // NW
CH = 32
NCHUNK = BPW // CH
SCH = 128
NSCHUNK = BPW // SCH
NBUF = 2


def _pipe_gather(base, nchunks, ch, streams, sg, sw):
    def fire(c):
        slot = c % NBUF
        isl = pl.ds(c * ch, ch)
        return [pltpu.async_copy(tab.at[idx.at[isl]], bufs[slot], sg[slot])
                for (tab, idx, bufs, _) in streams]

    def write(c):
        slot = c % NBUF
        osl = pl.ds(base + c * ch, ch)
        ws = [pltpu.make_async_copy(bufs[slot], out.at[osl], sw[slot])
              for (_, _, bufs, out) in streams]
        for w in ws:
            w.start()
        return ws

    gathers = {0: fire(0)}
    writes = {}
    for c in range(nchunks):
        if c >= NBUF - 1:
            for w in writes.pop(c - (NBUF - 1)):
                w.wait()
        if c + 1 < nchunks:
            gathers[c + 1] = fire(c + 1)
        for g in gathers.pop(c):
            g.wait()
        writes[c] = write(c)
    for c in sorted(writes):
        for w in writes.pop(c):
            w.wait()


def _sc_feat_body(pi_hbm, ni_hbm, features, pf_out, nf_out,
                  pi_v, ni_v,
                  pf0, pf1, nf0, nf1,
                  sg0, sg1, sw0, sw1):
    wid = lax.axis_index("c") * NS + lax.axis_index("s")
    base = wid * BPW
    pltpu.sync_copy(pi_hbm.at[pl.ds(base, BPW)], pi_v)
    pltpu.sync_copy(ni_hbm.at[pl.ds(base, BPW)], ni_v)
    _pipe_gather(
        base, NCHUNK, CH,
        [(features, pi_v, (pf0, pf1), pf_out),
         (features, ni_v, (nf0, nf1), nf_out)],
        (sg0, sg1), (sw0, sw1))


@functools.partial(
    pl.kernel,
    out_type=(
        jax.ShapeDtypeStruct((B, F), jnp.float32),
        jax.ShapeDtypeStruct((B, F), jnp.float32),
    ),
    mesh=plsc.VectorSubcoreMesh(core_axis_name="c", subcore_axis_name="s"),
    scratch_types=(
        [pltpu.VMEM((BPW,), jnp.int32)] * 2
        + [pltpu.VMEM((CH, F), jnp.float32)] * 4
        + [pltpu.SemaphoreType.DMA] * 4
    ),
)
def _sc_feat(*refs):
    _sc_feat_body(*refs)


def _sc_small_body(ui_hbm, pi_hbm, ni_hbm, gamma_users, gamma_items,
                   theta_users, beta_items,
                   gu_out, tu_out, gip_out, gin_out, bp_out, bn_out,
                   ui_v, pi_v, ni_v,
                   gu_v, tu_v, gip_v, gin_v, bp_v, bn_v,
                   ui_s, pi_s, ni_s, sem_s):
    wid = lax.axis_index("c") * NS + lax.axis_index("s")
    base = wid * BPW
    pltpu.sync_copy(ui_hbm.at[pl.ds(base, BPW)], ui_v)
    pltpu.sync_copy(pi_hbm.at[pl.ds(base, BPW)], pi_v)
    pltpu.sync_copy(ni_hbm.at[pl.ds(base, BPW)], ni_v)

    def small_chunk(c):
        def extract(g, _):
            st = c * SCH + g * 16
            u16 = ui_v[pl.ds(st, 16)]
            p16 = pi_v[pl.ds(st, 16)]
            n16 = ni_v[pl.ds(st, 16)]
            for l in range(16):
                row = g * 16 + l
                ui_s[row] = u16[l]
                pi_s[row] = p16[l]
                ni_s[row] = n16[l]
            return 0
        lax.fori_loop(0, SCH // 16, extract, 0)

        def fire(e, _):
            u = ui_s[e]
            p = pi_s[e]
            n = ni_s[e]
            pltpu.async_copy(gamma_users.at[pl.ds(u, 1)],
                             gu_v.at[pl.ds(e, 1)], sem_s)
            pltpu.async_copy(theta_users.at[pl.ds(u, 1)],
                             tu_v.at[pl.ds(e, 1)], sem_s)
            pltpu.async_copy(gamma_items.at[pl.ds(p, 1)],
                             gip_v.at[pl.ds(e, 1)], sem_s)
            pltpu.async_copy(gamma_items.at[pl.ds(n, 1)],
                             gin_v.at[pl.ds(e, 1)], sem_s)
            pltpu.async_copy(beta_items.at[pl.ds(p, 1)],
                             bp_v.at[pl.ds(e, 1)], sem_s)
            pltpu.async_copy(beta_items.at[pl.ds(n, 1)],
                             bn_v.at[pl.ds(e, 1)], sem_s)
            return 0
        lax.fori_loop(0, SCH, fire, 0)

        def drain(e, _):
            pltpu.make_async_copy(gamma_users.at[pl.ds(0, 1)],
                                  gu_v.at[pl.ds(e, 1)], sem_s).wait()
            pltpu.make_async_copy(theta_users.at[pl.ds(0, 1)],
                                  tu_v.at[pl.ds(e, 1)], sem_s).wait()
            pltpu.make_async_copy(gamma_items.at[pl.ds(0, 1)],
                                  gip_v.at[pl.ds(e, 1)], sem_s).wait()
            pltpu.make_async_copy(gamma_items.at[pl.ds(0, 1)],
                                  gin_v.at[pl.ds(e, 1)], sem_s).wait()
            pltpu.make_async_copy(beta_items.at[pl.ds(0, 1)],
                                  bp_v.at[pl.ds(e, 1)], sem_s).wait()
            pltpu.make_async_copy(beta_items.at[pl.ds(0, 1)],
                                  bn_v.at[pl.ds(e, 1)], sem_s).wait()
            return 0
        lax.fori_loop(0, SCH, drain, 0)

    for c in range(NSCHUNK):
        small_chunk(c)
        osl = pl.ds(base + c * SCH, SCH)
        pltpu.sync_copy(gu_v, gu_out.at[osl])
        pltpu.sync_copy(tu_v, tu_out.at[osl])
        pltpu.sync_copy(gip_v, gip_out.at[osl])
        pltpu.sync_copy(gin_v, gin_out.at[osl])
        pltpu.sync_copy(bp_v, bp_out.at[osl])
        pltpu.sync_copy(bn_v, bn_out.at[osl])


@functools.partial(
    pl.kernel,
    out_type=(
        jax.ShapeDtypeStruct((B, DG), jnp.float32),
        jax.ShapeDtypeStruct((B, DG), jnp.float32),
        jax.ShapeDtypeStruct((B, DG), jnp.float32),
        jax.ShapeDtypeStruct((B, DG), jnp.float32),
        jax.ShapeDtypeStruct((B, 1), jnp.float32),
        jax.ShapeDtypeStruct((B, 1), jnp.float32),
    ),
    mesh=plsc.VectorSubcoreMesh(core_axis_name="c", subcore_axis_name="s"),
    scratch_types=(
        [pltpu.VMEM((BPW,), jnp.int32)] * 3
        + [pltpu.VMEM((SCH, DG), jnp.float32)] * 4
        + [pltpu.VMEM((SCH, 1), jnp.float32)] * 2
        + [pltpu.SMEM((SCH,), jnp.int32)] * 3
        + [pltpu.SemaphoreType.DMA] * 1
    ),
)
def _sc_small(*refs):
    _sc_small_body(*refs)


BB = 2048


def _tc_combine_body(pf, nf, gu, tu, gip, gin, bp, bn, emb, vb, out):
    diff = pf[...] - nf[...]
    g = jnp.dot(diff, emb[...], preferred_element_type=jnp.float32)
    s_vis = jnp.sum(tu[...] * g, axis=1, keepdims=True)
    s_bias = jnp.dot(diff, vb[...], preferred_element_type=jnp.float32)
    s_lat = jnp.sum(gu[...] * (gip[...] - gin[...]), axis=1, keepdims=True)
    out[...] = bp[...] - bn[...] + s_lat + s_vis + s_bias


def _tc_combine(pf, nf, gu, tu, gip, gin, bp, bn, emb, vb):
    bspec_f = pl.BlockSpec((BB, F), lambda i: (i, 0))
    bspec_s = pl.BlockSpec((BB, DG), lambda i: (i, 0))
    bspec_1 = pl.BlockSpec((BB, 1), lambda i: (i, 0))
    return pl.pallas_call(
        _tc_combine_body,
        grid=(B // BB,),
        in_specs=[
            bspec_f, bspec_f, bspec_s, bspec_s, bspec_s, bspec_s,
            bspec_1, bspec_1,
            pl.BlockSpec((F, DG), lambda i: (0, 0)),
            pl.BlockSpec((F, 1), lambda i: (0, 0)),
        ],
        out_specs=bspec_1,
        out_shape=jax.ShapeDtypeStruct((B, 1), jnp.float32),
    )(pf, nf, gu, tu, gip, gin, bp, bn, emb, vb)[:, 0]


def kernel(ui, pi, ni, features, gamma_users, gamma_items, theta_users,
           embedding, beta_items, visual_bias):
    pf, nf = _sc_feat(pi, ni, features)
    ui_d, _ = lax.optimization_barrier((ui, pf))
    gu, tu, gip, gin, bp, bn = _sc_small(
        ui_d, pi, ni, gamma_users, gamma_items, theta_users, beta_items)
    return _tc_combine(pf, nf, gu, tu, gip, gin, bp, bn, embedding, visual_bias)


# speedup vs baseline: 1.1224x; 1.0252x over previous
import functools

import jax
import jax.numpy as jnp
from jax import lax
from jax.experimental import pallas as pl
from jax.experimental.pallas import tpu as pltpu
from jax.experimental.pallas import tpu_sc as plsc

B = 16384
F = 512
DG = 32
NV = 100000
NC = 2
NS = 16
NW = NC * NS
BPW = B // NW
CH = 32
NCHUNK = BPW // CH
SCH = 128
NSCHUNK = BPW // SCH
NBUF = 2


def _pipe_gather(base, nchunks, ch, streams, sg, sw):
    def fire(c):
        slot = c % NBUF
        isl = pl.ds(c * ch, ch)
        return [pltpu.async_copy(tab.at[idx.at[isl]], bufs[slot], sg[slot])
                for (tab, idx, bufs, _) in streams]

    def write(c):
        slot = c % NBUF
        osl = pl.ds(base + c * ch, ch)
        ws = [pltpu.make_async_copy(bufs[slot], out.at[osl], sw[slot])
              for (_, _, bufs, out) in streams]
        for w in ws:
            w.start()
        return ws

    gathers = {0: fire(0)}
    writes = {}
    for c in range(nchunks):
        if c >= NBUF - 1:
            for w in writes.pop(c - (NBUF - 1)):
                w.wait()
        if c + 1 < nchunks:
            gathers[c + 1] = fire(c + 1)
        for g in gathers.pop(c):
            g.wait()
        writes[c] = write(c)
    for c in sorted(writes):
        for w in writes.pop(c):
            w.wait()


def _sc_feat_body(pi_hbm, ni_hbm, features, pf_out, nf_out,
                  pi_v, ni_v,
                  pf0, pf1, nf0, nf1,
                  sg0, sg1, sw0, sw1):
    wid = lax.axis_index("c") * NS + lax.axis_index("s")
    base = wid * BPW
    pltpu.sync_copy(pi_hbm.at[pl.ds(base, BPW)], pi_v)
    pltpu.sync_copy(ni_hbm.at[pl.ds(base, BPW)], ni_v)
    _pipe_gather(
        base, NCHUNK, CH,
        [(features, pi_v, (pf0, pf1), pf_out),
         (features, ni_v, (nf0, nf1), nf_out)],
        (sg0, sg1), (sw0, sw1))


@functools.partial(
    pl.kernel,
    out_type=(
        jax.ShapeDtypeStruct((B, F), jnp.float32),
        jax.ShapeDtypeStruct((B, F), jnp.float32),
    ),
    mesh=plsc.VectorSubcoreMesh(core_axis_name="c", subcore_axis_name="s"),
    scratch_types=(
        [pltpu.VMEM((BPW,), jnp.int32)] * 2
        + [pltpu.VMEM((CH, F), jnp.float32)] * 4
        + [pltpu.SemaphoreType.DMA] * 4
    ),
)
def _sc_feat(*refs):
    _sc_feat_body(*refs)


def _row_dma_gather(base, idx_refs, idx_smem, streams, sem_s):
    def chunk(c):
        def extract(g, _):
            st = c * SCH + g * 16
            vecs = [r[pl.ds(st, 16)] for r in idx_refs]
            for l in range(16):
                row = g * 16 + l
                for s, v in zip(idx_smem, vecs):
                    s[row] = v[l]
            return 0
        lax.fori_loop(0, SCH // 16, extract, 0)

        def fire(e, _):
            scalars = [s[e] for s in idx_smem]
            for (tab, ip, buf, _) in streams:
                pltpu.async_copy(tab.at[pl.ds(scalars[ip], 1)],
                                 buf.at[pl.ds(e, 1)], sem_s)
            return 0
        lax.fori_loop(0, SCH, fire, 0)

        def drain(e, _):
            for (tab, _, buf, _) in streams:
                pltpu.make_async_copy(tab.at[pl.ds(0, 1)],
                                      buf.at[pl.ds(e, 1)], sem_s).wait()
            return 0
        lax.fori_loop(0, SCH, drain, 0)

    for c in range(NSCHUNK):
        chunk(c)
        osl = pl.ds(base + c * SCH, SCH)
        for (_, _, buf, out) in streams:
            pltpu.sync_copy(buf, out.at[osl])


def _sc_user_body(ui_hbm, gamma_users, theta_users,
                  gu_out, tu_out,
                  ui_v, gu_v, tu_v, ui_s, sem_s):
    wid = lax.axis_index("c") * NS + lax.axis_index("s")
    base = wid * BPW
    pltpu.sync_copy(ui_hbm.at[pl.ds(base, BPW)], ui_v)
    _row_dma_gather(base, [ui_v], [ui_s],
                    [(gamma_users, 0, gu_v, gu_out),
                     (theta_users, 0, tu_v, tu_out)], sem_s)


@functools.partial(
    pl.kernel,
    out_type=(
        jax.ShapeDtypeStruct((B, DG), jnp.float32),
        jax.ShapeDtypeStruct((B, DG), jnp.float32),
    ),
    mesh=plsc.VectorSubcoreMesh(core_axis_name="c", subcore_axis_name="s"),
    scratch_types=(
        [pltpu.VMEM((BPW,), jnp.int32)]
        + [pltpu.VMEM((SCH, DG), jnp.float32)] * 2
        + [pltpu.SMEM((SCH,), jnp.int32)]
        + [pltpu.SemaphoreType.DMA]
    ),
)
def _sc_user(*refs):
    _sc_user_body(*refs)


def _sc_item_body(pi_hbm, ni_hbm, gamma_items, beta_items,
                  gip_out, gin_out, bp_out, bn_out,
                  pi_v, ni_v, gip_v, gin_v, bp_v, bn_v, pi_s, ni_s, sem_s):
    wid = lax.axis_index("c") * NS + lax.axis_index("s")
    base = wid * BPW
    pltpu.sync_copy(pi_hbm.at[pl.ds(base, BPW)], pi_v)
    pltpu.sync_copy(ni_hbm.at[pl.ds(base, BPW)], ni_v)
    _row_dma_gather(base, [pi_v, ni_v], [pi_s, ni_s],
                    [(gamma_items, 0, gip_v, gip_out),
                     (gamma_items, 1, gin_v, gin_out),
                     (beta_items, 0, bp_v, bp_out),
                     (beta_items, 1, bn_v, bn_out)], sem_s)


@functools.partial(
    pl.kernel,
    out_type=(
        jax.ShapeDtypeStruct((B, DG), jnp.float32),
        jax.ShapeDtypeStruct((B, DG), jnp.float32),
        jax.ShapeDtypeStruct((B, 1), jnp.float32),
        jax.ShapeDtypeStruct((B, 1), jnp.float32),
    ),
    mesh=plsc.VectorSubcoreMesh(core_axis_name="c", subcore_axis_name="s"),
    scratch_types=(
        [pltpu.VMEM((BPW,), jnp.int32)] * 2
        + [pltpu.VMEM((SCH, DG), jnp.float32)] * 2
        + [pltpu.VMEM((SCH, 1), jnp.float32)] * 2
        + [pltpu.SMEM((SCH,), jnp.int32)] * 2
        + [pltpu.SemaphoreType.DMA]
    ),
)
def _sc_item(*refs):
    _sc_item_body(*refs)


BB = 2048


def _tc_combine_body(pf, nf, gu, tu, gip, gin, bp, bn, emb, vb, out):
    diff = pf[...] - nf[...]
    g = jnp.dot(diff, emb[...], preferred_element_type=jnp.float32)
    s_vis = jnp.sum(tu[...] * g, axis=1, keepdims=True)
    s_bias = jnp.dot(diff, vb[...], preferred_element_type=jnp.float32)
    s_lat = jnp.sum(gu[...] * (gip[...] - gin[...]), axis=1, keepdims=True)
    out[...] = bp[...] - bn[...] + s_lat + s_vis + s_bias


def _tc_combine(pf, nf, gu, tu, gip, gin, bp, bn, emb, vb):
    bspec_f = pl.BlockSpec((BB, F), lambda i: (i, 0))
    bspec_s = pl.BlockSpec((BB, DG), lambda i: (i, 0))
    bspec_1 = pl.BlockSpec((BB, 1), lambda i: (i, 0))
    return pl.pallas_call(
        _tc_combine_body,
        grid=(B // BB,),
        in_specs=[
            bspec_f, bspec_f, bspec_s, bspec_s, bspec_s, bspec_s,
            bspec_1, bspec_1,
            pl.BlockSpec((F, DG), lambda i: (0, 0)),
            pl.BlockSpec((F, 1), lambda i: (0, 0)),
        ],
        out_specs=bspec_1,
        out_shape=jax.ShapeDtypeStruct((B, 1), jnp.float32),
    )(pf, nf, gu, tu, gip, gin, bp, bn, emb, vb)[:, 0]


def kernel(ui, pi, ni, features, gamma_users, gamma_items, theta_users,
           embedding, beta_items, visual_bias):
    pf, nf = _sc_feat(pi, ni, features)
    ui_d, _ = lax.optimization_barrier((ui, pf))
    pi_d, _ = lax.optimization_barrier((pi, pf))
    gip, gin, bp, bn = _sc_item(pi_d, ni, gamma_items, beta_items)
    gu, tu = _sc_user(ui_d, gamma_users, theta_users)
    return _tc_combine(pf, nf, gu, tu, gip, gin, bp, bn, embedding, visual_bias)
